# baseline (speedup 1.0000x reference)
import functools

import jax
import jax.numpy as jnp
from jax import lax
from jax.experimental import pallas as pl
from jax.experimental.pallas import tpu as pltpu
from jax.experimental.pallas import tpu_sc as plsc
from jax.experimental.layout import Layout, with_layout_constraint

NC = 2
NS = 16
NW = NC * NS
G = 128
NBUF = 2


@functools.lru_cache(maxsize=None)
def _build(seq: int, bsz: int, d: int):
  mesh = plsc.VectorSubcoreMesh(
      core_axis_name="c", subcore_axis_name="s",
      num_cores=NC, num_subcores=NS)

  ht = d // 8
  tw = 8 * G
  per_w = (bsz // NW) * seq

  @functools.partial(
      pl.kernel,
      out_type=jax.ShapeDtypeStruct((seq, ht, (bsz // G) * tw), jnp.float32),
      mesh=mesh,
      scratch_types=[
          pltpu.VMEM((per_w,), jnp.int32),
          pltpu.VMEM((NBUF, G), jnp.int32),
          pltpu.VMEM((NBUF, G, d), jnp.float32),
          pltpu.VMEM((NBUF, d * G), jnp.float32),
          pltpu.SemaphoreType.DMA((NBUF,)),
          pltpu.SemaphoreType.DMA((NBUF,)),
      ],
      compiler_params=pltpu.CompilerParams(
          use_tc_tiling_on_sc=False, needs_layout_passes=False),
  )
  def gather_kernel(ids_hbm, emb_hbm, out_hbm, idx_chunk, idx_stage,
                    rows_v, tr_v, gsem, osem):
    wid = lax.axis_index("s") * NC + lax.axis_index("c")

    pltpu.sync_copy(ids_hbm.at[wid], idx_chunk)

    jvecs = [jnp.arange(16, dtype=jnp.int32) + 16 * k for k in range(8)]

    def repack(s, b):
      vals = [plsc.load_gather(idx_chunk, [jvecs[k] * seq + s])
              for k in range(G // 16)]
      for k in range(G // 16):
        idx_stage[b, pl.ds(16 * k, 16)] = vals[k]

    lvec = jnp.arange(16, dtype=jnp.int32)
    rot = [(lvec + o) & 15 for o in range(16)]
    st_off = [r * G + lvec for r in rot]

    def transpose_block(b):
      rows = rows_v.at[b]
      tr = tr_v.at[b]

      @pl.loop(0, d // 16)
      def _m(m):
        hb = 16 * m
        hrot = [hb + rot[o] for o in range(16)]
        for k in range(G // 16):
          base = hb * G + 16 * k
          vals = {}
          for o in range(16):
            vals[o] = plsc.load_gather(rows, [jvecs[k], hrot[o]])
            if o >= 8:
              plsc.store_scatter(tr, [st_off[o - 8] + base], vals[o - 8])
          for o in range(8, 16):
            plsc.store_scatter(tr, [st_off[o] + base], vals[o])

    def start_gather(b):
      pltpu.async_copy(emb_hbm.at[idx_stage.at[b]], rows_v.at[b], gsem.at[b])

    def wait_gather(b):
      pltpu.make_async_copy(
          emb_hbm.at[idx_stage.at[b]], rows_v.at[b], gsem.at[b]).wait()

    def start_write(s, b):
      for t in range(ht):
        pltpu.async_copy(tr_v.at[b, pl.ds(t * tw, tw)],
                         out_hbm.at[s, t, pl.ds(wid * tw, tw)],
                         osem.at[b])

    def wait_write(s, b):
      for t in range(ht):
        pltpu.make_async_copy(
            tr_v.at[b, pl.ds(t * tw, tw)],
            out_hbm.at[s, t, pl.ds(wid * tw, tw)],
            osem.at[b]).wait()

    for b in range(NBUF):
      repack(b, b)
      start_gather(b)

    for b in range(NBUF):
      wait_gather(b)
      transpose_block(b)
      repack(b + NBUF, b)
      start_gather(b)
      start_write(b, b)

    @pl.loop(NBUF, seq - NBUF, step=NBUF)
    def _steady(s0):
      for b in range(NBUF):
        s = s0 + b
        wait_write(s, b)
        wait_gather(b)
        transpose_block(b)
        repack(s + NBUF, b)
        start_gather(b)
        start_write(s, b)

    for b in range(NBUF):
      s = seq - NBUF + b
      wait_write(s, b)
      wait_gather(b)
      transpose_block(b)
      start_write(s, b)
    for b in range(NBUF):
      wait_write(seq - NBUF + b, b)

  return gather_kernel


def kernel(input_ids, emb):
  bsz, seq = input_ids.shape
  _, d = emb.shape
  ids2 = input_ids.astype(jnp.int32).reshape(NW, (bsz // NW) * seq)
  out_t = _build(seq, bsz, d)(ids2, emb)
  out5 = out_t.reshape(seq, d // 8, bsz // G, 8, G)
  out5 = with_layout_constraint(
      out5, Layout(major_to_minor=(0, 1, 2, 3, 4)))
  return out5.transpose(2, 4, 0, 1, 3).reshape(bsz, seq, d)


# speedup vs baseline: 1.0176x; 1.0176x over previous
import functools

import jax
import jax.numpy as jnp
from jax import lax
from jax.experimental import pallas as pl
from jax.experimental.pallas import tpu as pltpu
from jax.experimental.pallas import tpu_sc as plsc
from jax.experimental.layout import Layout, with_layout_constraint

NC = 2
NS = 16
NW = NC * NS
G = 128
NBUF = 2


@functools.lru_cache(maxsize=None)
def _build(seq: int, bsz: int, d: int):
  mesh = plsc.VectorSubcoreMesh(
      core_axis_name="c", subcore_axis_name="s",
      num_cores=NC, num_subcores=NS)

  ht = d // 8
  tw = 8 * G
  per_w = (bsz // NW) * seq

  @functools.partial(
      pl.kernel,
      out_type=jax.ShapeDtypeStruct((seq, ht, (bsz // G) * tw), jnp.float32),
      mesh=mesh,
      scratch_types=[
          pltpu.VMEM((per_w,), jnp.int32),
          pltpu.VMEM((NBUF, G), jnp.int32),
          pltpu.VMEM((NBUF, G, d), jnp.float32),
          pltpu.VMEM((NBUF, d * G), jnp.float32),
          pltpu.SemaphoreType.DMA((NBUF,)),
          pltpu.SemaphoreType.DMA((NBUF,)),
      ],
      compiler_params=pltpu.CompilerParams(
          use_tc_tiling_on_sc=False, needs_layout_passes=False),
  )
  def gather_kernel(ids_hbm, emb_hbm, out_hbm, idx_chunk, idx_stage,
                    rows_v, tr_v, gsem, osem):
    wid = lax.axis_index("s") * NC + lax.axis_index("c")

    pltpu.sync_copy(ids_hbm.at[wid], idx_chunk)

    jvecs = [jnp.arange(16, dtype=jnp.int32) + 16 * k for k in range(8)]

    def repack(s, b):
      vals = [plsc.load_gather(idx_chunk, [jvecs[k] * seq + s])
              for k in range(G // 16)]
      for k in range(G // 16):
        idx_stage[b, pl.ds(16 * k, 16)] = vals[k]

    lvec = jnp.arange(16, dtype=jnp.int32)
    rot = [(lvec + o) & 15 for o in range(16)]
    st_off = [r * G + lvec for r in rot]

    def transpose_block(b):
      rows = rows_v.at[b]
      tr = tr_v.at[b]

      @pl.loop(0, d // 16)
      def _m(m):
        hb = 16 * m
        for k in range(G // 16):
          vals = [plsc.load_gather(rows, [jvecs[k], hb + rot[o]])
                  for o in range(16)]
          for o in range(16):
            plsc.store_scatter(tr, [st_off[o] + (hb * G + 16 * k)], vals[o])

    def start_gather(b):
      pltpu.async_copy(emb_hbm.at[idx_stage.at[b]], rows_v.at[b], gsem.at[b])

    def wait_gather(b):
      pltpu.make_async_copy(
          emb_hbm.at[idx_stage.at[b]], rows_v.at[b], gsem.at[b]).wait()

    def start_write(s, b):
      for t in range(ht):
        pltpu.async_copy(tr_v.at[b, pl.ds(t * tw, tw)],
                         out_hbm.at[s, t, pl.ds(wid * tw, tw)],
                         osem.at[b])

    def wait_write(s, b):
      for t in range(ht):
        pltpu.make_async_copy(
            tr_v.at[b, pl.ds(t * tw, tw)],
            out_hbm.at[s, t, pl.ds(wid * tw, tw)],
            osem.at[b]).wait()

    for b in range(NBUF):
      repack(b, b)
      start_gather(b)

    for b in range(NBUF):
      wait_gather(b)
      transpose_block(b)
      repack(b + NBUF, b)
      start_gather(b)
      start_write(b, b)

    @pl.loop(NBUF, seq - NBUF, step=NBUF)
    def _steady(s0):
      for b in range(NBUF):
        s = s0 + b
        wait_write(s, b)
        wait_gather(b)
        transpose_block(b)
        repack(s + NBUF, b)
        start_gather(b)
        start_write(s, b)

    for b in range(NBUF):
      s = seq - NBUF + b
      wait_write(s, b)
      wait_gather(b)
      transpose_block(b)
      start_write(s, b)
    for b in range(NBUF):
      wait_write(seq - NBUF + b, b)

  return gather_kernel


def kernel(input_ids, emb):
  bsz, seq = input_ids.shape
  _, d = emb.shape
  ids2 = input_ids.astype(jnp.int32).reshape(NW, (bsz // NW) * seq)
  out_t = _build(seq, bsz, d)(ids2, emb)
  out5 = out_t.reshape(seq, d // 8, bsz // G, 8, G)
  out5 = with_layout_constraint(
      out5, Layout(major_to_minor=(0, 1, 2, 3, 4)))
  return out5.transpose(2, 4, 0, 1, 3).reshape(bsz, seq, d)


# speedup vs baseline: 1.0269x; 1.0092x over previous
import functools

import jax
import jax.numpy as jnp
from jax import lax
from jax.experimental import pallas as pl
from jax.experimental.pallas import tpu as pltpu
from jax.experimental.pallas import tpu_sc as plsc
from jax.experimental.layout import Layout, with_layout_constraint

NC = 2
NS = 16
NW = NC * NS
G = 128
NBUF = 4


@functools.lru_cache(maxsize=None)
def _build(seq: int, bsz: int, d: int):
  mesh = plsc.VectorSubcoreMesh(
      core_axis_name="c", subcore_axis_name="s",
      num_cores=NC, num_subcores=NS)

  ht = d // 8
  tw = 8 * G
  per_w = (bsz // NW) * seq

  @functools.partial(
      pl.kernel,
      out_type=jax.ShapeDtypeStruct((seq, ht, (bsz // G) * tw), jnp.float32),
      mesh=mesh,
      scratch_types=[
          pltpu.VMEM((per_w,), jnp.int32),
          pltpu.VMEM((NBUF, G), jnp.int32),
          pltpu.VMEM((NBUF, G, d), jnp.float32),
          pltpu.VMEM((NBUF, d * G), jnp.float32),
          pltpu.SemaphoreType.DMA((NBUF,)),
          pltpu.SemaphoreType.DMA((NBUF,)),
      ],
      compiler_params=pltpu.CompilerParams(
          use_tc_tiling_on_sc=False, needs_layout_passes=False),
  )
  def gather_kernel(ids_hbm, emb_hbm, out_hbm, idx_chunk, idx_stage,
                    rows_v, tr_v, gsem, osem):
    wid = lax.axis_index("s") * NC + lax.axis_index("c")

    pltpu.sync_copy(ids_hbm.at[wid], idx_chunk)

    jvecs = [jnp.arange(16, dtype=jnp.int32) + 16 * k for k in range(8)]

    def repack(s, b):
      vals = [plsc.load_gather(idx_chunk, [jvecs[k] * seq + s])
              for k in range(G // 16)]
      for k in range(G // 16):
        idx_stage[b, pl.ds(16 * k, 16)] = vals[k]

    lvec = jnp.arange(16, dtype=jnp.int32)
    rot = [(lvec + o) & 15 for o in range(16)]
    st_off = [r * G + lvec for r in rot]

    def transpose_block(b):
      rows = rows_v.at[b]
      tr = tr_v.at[b]

      @pl.loop(0, d // 16)
      def _m(m):
        hb = 16 * m
        for k in range(G // 16):
          vals = [plsc.load_gather(rows, [jvecs[k], hb + rot[o]])
                  for o in range(16)]
          for o in range(16):
            plsc.store_scatter(tr, [st_off[o] + (hb * G + 16 * k)], vals[o])

    def start_gather(b):
      pltpu.async_copy(emb_hbm.at[idx_stage.at[b]], rows_v.at[b], gsem.at[b])

    def wait_gather(b):
      pltpu.make_async_copy(
          emb_hbm.at[idx_stage.at[b]], rows_v.at[b], gsem.at[b]).wait()

    def start_write(s, b):
      for t in range(ht):
        pltpu.async_copy(tr_v.at[b, pl.ds(t * tw, tw)],
                         out_hbm.at[s, t, pl.ds(wid * tw, tw)],
                         osem.at[b])

    def wait_write(s, b):
      for t in range(ht):
        pltpu.make_async_copy(
            tr_v.at[b, pl.ds(t * tw, tw)],
            out_hbm.at[s, t, pl.ds(wid * tw, tw)],
            osem.at[b]).wait()

    for b in range(NBUF):
      repack(b, b)
      start_gather(b)

    for b in range(NBUF):
      wait_gather(b)
      transpose_block(b)
      repack(b + NBUF, b)
      start_gather(b)
      start_write(b, b)

    @pl.loop(NBUF, seq - NBUF, step=NBUF)
    def _steady(s0):
      for b in range(NBUF):
        s = s0 + b
        wait_write(s, b)
        wait_gather(b)
        transpose_block(b)
        repack(s + NBUF, b)
        start_gather(b)
        start_write(s, b)

    for b in range(NBUF):
      s = seq - NBUF + b
      wait_write(s, b)
      wait_gather(b)
      transpose_block(b)
      start_write(s, b)
    for b in range(NBUF):
      wait_write(seq - NBUF + b, b)

  return gather_kernel


def kernel(input_ids, emb):
  bsz, seq = input_ids.shape
  _, d = emb.shape
  ids2 = input_ids.astype(jnp.int32).reshape(NW, (bsz // NW) * seq)
  out_t = _build(seq, bsz, d)(ids2, emb)
  out5 = out_t.reshape(seq, d // 8, bsz // G, 8, G)
  out5 = with_layout_constraint(
      out5, Layout(major_to_minor=(0, 1, 2, 3, 4)))
  return out5.transpose(2, 4, 0, 1, 3).reshape(bsz, seq, d)
